# baseline (speedup 1.0000x reference)
import functools

import jax
import jax.numpy as jnp
from jax import lax
from jax.experimental import pallas as pl
from jax.experimental.pallas import tpu as pltpu
from jax.experimental.pallas import tpu_sc as plsc

_B = 16384
_D = 32
_L = 16
_NC, _NS = 2, 16
_NW = _NC * _NS
_RPW = _B // _NW
_CH = 128
_NCH = _RPW // _CH
_GR = 4
_V = 1000000 * _D // 128
_MAX_NORM = 2.0


def _rsqrt(x):
    i = plsc.bitcast(x, jnp.int32)
    i = jnp.int32(0x5F3759DF) - (i >> 1)
    y = plsc.bitcast(i, jnp.float32)
    for _ in range(3):
        y = y * (jnp.float32(1.5) - jnp.float32(0.5) * x * y * y)
    return y


def _scale(n2):
    y = _rsqrt(jnp.maximum(n2, jnp.float32(1e-12)))
    return jnp.minimum(jnp.float32(1.0), jnp.float32(_MAX_NORM) * y)


_mesh = plsc.VectorSubcoreMesh(core_axis_name="c", subcore_axis_name="s")


@functools.partial(
    pl.kernel,
    mesh=_mesh,
    out_type=jax.ShapeDtypeStruct((_B,), jnp.float32),
    compiler_params=pltpu.CompilerParams(
        needs_layout_passes=False, use_tc_tiling_on_sc=True),
    scratch_types=[
        pltpu.VMEM((_RPW,), jnp.int32),
        pltpu.VMEM((_RPW,), jnp.int32),
        pltpu.VMEM((2, _CH), jnp.int32),
        pltpu.VMEM((2, _CH), jnp.int32),
        pltpu.VMEM((2, _CH, 128), jnp.float32),
        pltpu.VMEM((2, _CH, 128), jnp.float32),
        pltpu.VMEM((_RPW,), jnp.float32),
        pltpu.SemaphoreType.DMA,
        pltpu.SemaphoreType.DMA,
    ],
)
def _als_logits(u_hbm, v_hbm, users_hbm, items_hbm, out_hbm,
                uidx, vidx, uq, vq, ubuf, vbuf, outv, usem, vsem):
    wid = lax.axis_index("s") * _NC + lax.axis_index("c")
    base = wid * _RPW
    pltpu.sync_copy(u_hbm.at[pl.ds(base, _RPW)], uidx)
    pltpu.sync_copy(v_hbm.at[pl.ds(base, _RPW)], vidx)

    def build_lists(ch, slot):
        for k in range(_CH // _L):
            uvec = uidx[pl.ds(ch * _CH + k * _L, _L)]
            vvec = vidx[pl.ds(ch * _CH + k * _L, _L)]
            uq[slot, pl.ds(k * _L, _L)] = uvec >> 2
            vq[slot, pl.ds(k * _L, _L)] = vvec >> 2

    def fire(ch, slot):
        cu = pltpu.async_copy(users_hbm.at[uq.at[slot]], ubuf.at[slot], usem)
        cv = pltpu.async_copy(items_hbm.at[vq.at[slot]], vbuf.at[slot], vsem)
        return cu, cv

    def compute_chunk(ch, slot):
        for g in range(_CH // _L):
            ri = g * _L + lax.iota(jnp.int32, _L)
            su = (uidx[pl.ds(ch * _CH + g * _L, _L)] & 3) * _D
            sv = (vidx[pl.ds(ch * _CH + g * _L, _L)] & 3) * _D
            uv = jnp.zeros((_L,), jnp.float32)
            uu = jnp.zeros((_L,), jnp.float32)
            vv = jnp.zeros((_L,), jnp.float32)
            for d in range(_D):
                a = plsc.load_gather(ubuf, [jnp.full((_L,), slot, jnp.int32),
                                            ri, su + d])
                b = plsc.load_gather(vbuf, [jnp.full((_L,), slot, jnp.int32),
                                            ri, sv + d])
                uv = uv + a * b
                uu = uu + a * a
                vv = vv + b * b
            out_off = ch * _CH + g * _L
            outv[pl.ds(out_off, _L)] = uv * _scale(uu) * _scale(vv)

    build_lists(0, 0)
    cps = fire(0, 0)
    for ch in range(_NCH):
        nxt = None
        if ch + 1 < _NCH:
            build_lists(ch + 1, (ch + 1) % 2)
            nxt = fire(ch + 1, (ch + 1) % 2)
        cps[0].wait()
        cps[1].wait()
        compute_chunk(ch, ch % 2)
        cps = nxt

    pltpu.sync_copy(outv, out_hbm.at[pl.ds(base, _RPW)])


def kernel(u, v, users_table, items_table):
    users2 = users_table.reshape(_V, 128)
    items2 = items_table.reshape(_V, 128)
    return _als_logits(u.astype(jnp.int32), v.astype(jnp.int32),
                       users2, items2)


# speedup vs baseline: 1.3681x; 1.3681x over previous
import functools

import jax
import jax.numpy as jnp
from jax import lax
from jax.experimental import pallas as pl
from jax.experimental.pallas import tpu as pltpu
from jax.experimental.pallas import tpu_sc as plsc

_B = 16384
_D = 32
_L = 16
_NC, _NS = 2, 16
_NW = _NC * _NS
_RPW = _B // _NW
_NCHUNK = _RPW // _L
_MAX_NORM = 2.0


def _rsqrt(x):
    i = plsc.bitcast(x, jnp.int32)
    i = jnp.int32(0x5F3759DF) - (i >> 1)
    y = plsc.bitcast(i, jnp.float32)
    for _ in range(3):
        y = y * (jnp.float32(1.5) - jnp.float32(0.5) * x * y * y)
    return y


def _scale(n2):
    y = _rsqrt(jnp.maximum(n2, jnp.float32(1e-12)))
    return jnp.minimum(jnp.float32(1.0), jnp.float32(_MAX_NORM) * y)


_mesh = plsc.VectorSubcoreMesh(core_axis_name="c", subcore_axis_name="s")


@functools.partial(
    pl.kernel,
    mesh=_mesh,
    out_type=jax.ShapeDtypeStruct((_B,), jnp.float32),
    compiler_params=pltpu.CompilerParams(
        needs_layout_passes=False, use_tc_tiling_on_sc=True),
    scratch_types=[
        pltpu.VMEM((_RPW,), jnp.int32),
        pltpu.VMEM((_RPW,), jnp.int32),
        pltpu.VMEM((2, _L, 8, _D), jnp.float32),
        pltpu.VMEM((2, _L, 8, _D), jnp.float32),
        pltpu.VMEM((_RPW * _D,), jnp.float32),
        pltpu.VMEM((_RPW * _D,), jnp.float32),
        pltpu.VMEM((_RPW,), jnp.float32),
        pltpu.SemaphoreType.DMA,
        pltpu.SemaphoreType.DMA,
        pltpu.SemaphoreType.DMA,
        pltpu.SemaphoreType.DMA,
    ],
)
def _als_logits(u_hbm, v_hbm, users_hbm, items_hbm, out_hbm,
                uidx, vidx, uring, vring, ue, ve, outv,
                usemA, vsemA, usemB, vsemB):
    wid = lax.axis_index("s") * _NC + lax.axis_index("c")
    base = wid * _RPW
    pltpu.sync_copy(u_hbm.at[pl.ds(base, _RPW)], uidx)
    pltpu.sync_copy(v_hbm.at[pl.ds(base, _RPW)], vidx)

    def issue(ch, pbuf, usem, vsem):
        uvec = uidx[pl.ds(ch * _L, _L)]
        vvec = vidx[pl.ds(ch * _L, _L)]
        ucps, vcps = [], []
        for j in range(_L):
            gu = pl.multiple_of((uvec[j] >> 3) * 8, 8)
            gv = pl.multiple_of((vvec[j] >> 3) * 8, 8)
            ucps.append(pltpu.async_copy(
                users_hbm.at[pl.ds(gu, 8), :], uring.at[pbuf, j], usem))
            vcps.append(pltpu.async_copy(
                items_hbm.at[pl.ds(gv, 8), :], vring.at[pbuf, j], vsem))
        return ucps, vcps

    def extract(ch, pbuf, cps):
        ucps, vcps = cps
        for cp in ucps:
            cp.wait()
        for cp in vcps:
            cp.wait()
        uvec = uidx[pl.ds(ch * _L, _L)]
        vvec = vidx[pl.ds(ch * _L, _L)]
        for j in range(_L):
            i = ch * _L + j
            ru = uvec[j] & 7
            ue[pl.ds(i * _D, _L)] = uring[pbuf, j, ru, pl.ds(0, _L)]
            ue[pl.ds(i * _D + _L, _L)] = uring[pbuf, j, ru, pl.ds(_L, _L)]
            rv = vvec[j] & 7
            ve[pl.ds(i * _D, _L)] = vring[pbuf, j, rv, pl.ds(0, _L)]
            ve[pl.ds(i * _D + _L, _L)] = vring[pbuf, j, rv, pl.ds(_L, _L)]

    def pipelined(k, carry):
        cpsA = issue(2 * k, 0, usemA, vsemA)
        cpsB = issue(2 * k + 1, 1, usemB, vsemB)
        extract(2 * k, 0, cpsA)
        extract(2 * k + 1, 1, cpsB)
        return carry

    lax.fori_loop(0, _NCHUNK // 2, pipelined, 0)

    def group(g, carry):
        flat = (g * _L + lax.iota(jnp.int32, _L)) * _D
        uv = jnp.zeros((_L,), jnp.float32)
        uu = jnp.zeros((_L,), jnp.float32)
        vv = jnp.zeros((_L,), jnp.float32)
        for d in range(_D):
            a = plsc.load_gather(ue, [flat + d])
            b = plsc.load_gather(ve, [flat + d])
            uv = uv + a * b
            uu = uu + a * a
            vv = vv + b * b
        outv[pl.ds(g * _L, _L)] = uv * _scale(uu) * _scale(vv)
        return carry

    lax.fori_loop(0, _NCHUNK, group, 0)
    pltpu.sync_copy(outv, out_hbm.at[pl.ds(base, _RPW)])


def kernel(u, v, users_table, items_table):
    return _als_logits(u.astype(jnp.int32), v.astype(jnp.int32),
                       users_table, items_table)


# speedup vs baseline: 1.4107x; 1.0311x over previous
import functools

import jax
import jax.numpy as jnp
from jax import lax
from jax.experimental import pallas as pl
from jax.experimental.pallas import tpu as pltpu
from jax.experimental.pallas import tpu_sc as plsc

_B = 16384
_D = 32
_L = 16
_NC, _NS = 2, 16
_NW = _NC * _NS
_RPW = _B // _NW
_NCHUNK = _RPW // _L
_MAX_NORM = 2.0

_params = pltpu.CompilerParams(
    needs_layout_passes=False, use_tc_tiling_on_sc=True)
_mesh = plsc.VectorSubcoreMesh(core_axis_name="c", subcore_axis_name="s")


def _rsqrt(x):
    i = plsc.bitcast(x, jnp.int32)
    i = jnp.int32(0x5F3759DF) - (i >> 1)
    y = plsc.bitcast(i, jnp.float32)
    for _ in range(3):
        y = y * (jnp.float32(1.5) - jnp.float32(0.5) * x * y * y)
    return y


def _scale(n2):
    y = _rsqrt(jnp.maximum(n2, jnp.float32(1e-12)))
    return jnp.minimum(jnp.float32(1.0), jnp.float32(_MAX_NORM) * y)


def _fetch_rows(t_hbm, idx, ring, flat, semA, semB):
    def issue(ch, pbuf, sem):
        vec = idx[pl.ds(ch * _L, _L)]
        cps = []
        for j in range(_L):
            g = pl.multiple_of((vec[j] >> 3) * 8, 8)
            cps.append(pltpu.async_copy(
                t_hbm.at[pl.ds(g, 8), :], ring.at[pbuf, j], sem))
        return cps

    def extract(ch, pbuf, cps):
        for cp in cps:
            cp.wait()
        vec = idx[pl.ds(ch * _L, _L)]
        for j in range(_L):
            i = ch * _L + j
            r = vec[j] & 7
            flat[pl.ds(i * _D, _L)] = ring[pbuf, j, r, pl.ds(0, _L)]
            flat[pl.ds(i * _D + _L, _L)] = ring[pbuf, j, r, pl.ds(_L, _L)]

    def body(k, carry):
        cpsA = issue(2 * k, 0, semA)
        cpsB = issue(2 * k + 1, 1, semB)
        extract(2 * k, 0, cpsA)
        extract(2 * k + 1, 1, cpsB)
        return carry

    lax.fori_loop(0, _NCHUNK // 2, body, 0)


@functools.partial(
    pl.kernel,
    mesh=_mesh,
    out_type=(jax.ShapeDtypeStruct((_B * _D,), jnp.float32),
              jax.ShapeDtypeStruct((_B,), jnp.float32)),
    compiler_params=_params,
    scratch_types=[
        pltpu.VMEM((_RPW,), jnp.int32),
        pltpu.VMEM((2, _L, 8, _D), jnp.float32),
        pltpu.VMEM((_RPW * _D,), jnp.float32),
        pltpu.VMEM((_RPW,), jnp.float32),
        pltpu.SemaphoreType.DMA,
        pltpu.SemaphoreType.DMA,
    ],
)
def _user_rows(u_hbm, users_hbm, rows_out, su_out,
               uidx, uring, ue, sv, semA, semB):
    wid = lax.axis_index("s") * _NC + lax.axis_index("c")
    base = wid * _RPW
    pltpu.sync_copy(u_hbm.at[pl.ds(base, _RPW)], uidx)
    _fetch_rows(users_hbm, uidx, uring, ue, semA, semB)

    def group(g, carry):
        flat = (g * _L + lax.iota(jnp.int32, _L)) * _D
        uu = jnp.zeros((_L,), jnp.float32)
        for d in range(_D):
            a = plsc.load_gather(ue, [flat + d])
            uu = uu + a * a
        sv[pl.ds(g * _L, _L)] = _scale(uu)
        return carry

    lax.fori_loop(0, _NCHUNK, group, 0)
    pltpu.sync_copy(ue, rows_out.at[pl.ds(base * _D, _RPW * _D)])
    pltpu.sync_copy(sv, su_out.at[pl.ds(base, _RPW)])


@functools.partial(
    pl.kernel,
    mesh=_mesh,
    out_type=jax.ShapeDtypeStruct((_B,), jnp.float32),
    compiler_params=_params,
    scratch_types=[
        pltpu.VMEM((_RPW,), jnp.int32),
        pltpu.VMEM((2, _L, 8, _D), jnp.float32),
        pltpu.VMEM((_RPW * _D,), jnp.float32),
        pltpu.VMEM((_RPW * _D,), jnp.float32),
        pltpu.VMEM((_RPW,), jnp.float32),
        pltpu.VMEM((_RPW,), jnp.float32),
        pltpu.SemaphoreType.DMA,
        pltpu.SemaphoreType.DMA,
    ],
)
def _combine(v_hbm, items_hbm, rows_hbm, su_hbm, out_hbm,
             vidx, vring, ve, ue, su, outv, semA, semB):
    wid = lax.axis_index("s") * _NC + lax.axis_index("c")
    base = wid * _RPW
    pltpu.sync_copy(v_hbm.at[pl.ds(base, _RPW)], vidx)
    pltpu.sync_copy(rows_hbm.at[pl.ds(base * _D, _RPW * _D)], ue)
    pltpu.sync_copy(su_hbm.at[pl.ds(base, _RPW)], su)
    _fetch_rows(items_hbm, vidx, vring, ve, semA, semB)

    def group(g, carry):
        flat = (g * _L + lax.iota(jnp.int32, _L)) * _D
        uv = jnp.zeros((_L,), jnp.float32)
        vv = jnp.zeros((_L,), jnp.float32)
        for d in range(_D):
            a = plsc.load_gather(ue, [flat + d])
            b = plsc.load_gather(ve, [flat + d])
            uv = uv + a * b
            vv = vv + b * b
        s_u = su[pl.ds(g * _L, _L)]
        outv[pl.ds(g * _L, _L)] = uv * s_u * _scale(vv)
        return carry

    lax.fori_loop(0, _NCHUNK, group, 0)
    pltpu.sync_copy(outv, out_hbm.at[pl.ds(base, _RPW)])


def kernel(u, v, users_table, items_table):
    rows, su = _user_rows(u.astype(jnp.int32), users_table)
    return _combine(v.astype(jnp.int32), items_table, rows, su)
